# baseline (device time: 19232 ns/iter reference)
import jax
import jax.numpy as jnp
from jax import lax
from jax.experimental import pallas as pl
from jax.experimental.pallas import tpu as pltpu

N_DEV = 8


def kernel(x, w_mat):
    m_per, k = x.shape
    _, n_per = w_mat.shape
    kh = k // 2

    def body(x_ref, w_ref, out_ref, xg_ref, w_bf,
             z_s, z_r, cw_s, cw_r, ccw_s, ccw_r):
        my = lax.axis_index("i")
        q = lax.rem(my, 4)
        zbase = my - q
        right = zbase + lax.rem(q + 1, 4)
        left = zbase + lax.rem(q + 3, 4)
        o2 = zbase + lax.rem(q + 2, 4)
        partner = lax.rem(my + 4, N_DEV)
        leftp = lax.rem(left + 4, N_DEV)
        rightp = lax.rem(right + 4, N_DEV)
        o2p = lax.rem(o2 + 4, N_DEV)

        barrier_sem = pltpu.get_barrier_semaphore()
        for nbr in (left, right, partner):
            pl.semaphore_signal(
                barrier_sem, inc=1,
                device_id=(nbr,), device_id_type=pl.DeviceIdType.MESH,
            )
        xg_ref[pl.ds(my * m_per, m_per), :] = x_ref[...].astype(jnp.bfloat16)
        pl.semaphore_wait(barrier_sem, 3)

        def full(org):
            return xg_ref.at[pl.ds(org * m_per, m_per), :]

        def half(org, h):
            return xg_ref.at[pl.ds(org * m_per, m_per), pl.ds(h * kh, kh)]

        sends = []

        def send(src, dst_dev, ssem, rsem):
            rdma = pltpu.make_async_remote_copy(
                src_ref=src, dst_ref=src,
                send_sem=ssem, recv_sem=rsem,
                device_id=(dst_dev,), device_id_type=pl.DeviceIdType.MESH,
            )
            rdma.start()
            sends.append(rdma)

        def wait_recv(dst, rsem):
            recv = pltpu.make_async_remote_copy(
                src_ref=dst, dst_ref=dst,
                send_sem=z_s.at[0], recv_sem=rsem,
                device_id=(my,), device_id_type=pl.DeviceIdType.MESH,
            )
            recv.wait_recv()

        def chunk_gemm(org):
            out_ref[pl.ds(org * m_per, m_per), :] = jnp.dot(
                xg_ref[pl.ds(org * m_per, m_per), :], w_bf[...],
                preferred_element_type=jnp.float32,
            )

        send(full(my), partner, z_s.at[0], z_r.at[0])
        send(full(my), right, cw_s.at[0], cw_r.at[0])
        send(full(my), left, ccw_s.at[0], ccw_r.at[0])

        w_bf[...] = w_ref[...].astype(jnp.bfloat16)
        chunk_gemm(my)

        wait_recv(full(partner), z_r.at[0])
        send(half(partner, 1), right, cw_s.at[1], cw_r.at[1])
        send(half(partner, 0), left, ccw_s.at[1], ccw_r.at[1])
        chunk_gemm(partner)

        wait_recv(full(left), cw_r.at[0])
        send(half(left, 0), partner, z_s.at[1], z_r.at[1])
        send(half(left, 0), right, cw_s.at[2], cw_r.at[2])
        chunk_gemm(left)

        wait_recv(full(right), ccw_r.at[0])
        send(half(right, 1), partner, z_s.at[2], z_r.at[2])
        send(half(right, 1), left, ccw_s.at[2], ccw_r.at[2])
        chunk_gemm(right)

        wait_recv(half(leftp, 0), z_r.at[1])
        send(half(leftp, 0), right, cw_s.at[3], cw_r.at[3])

        wait_recv(half(leftp, 1), cw_r.at[1])
        chunk_gemm(leftp)

        wait_recv(half(rightp, 0), ccw_r.at[1])

        wait_recv(half(rightp, 1), z_r.at[2])
        send(half(rightp, 1), left, ccw_s.at[3], ccw_r.at[3])
        chunk_gemm(rightp)

        wait_recv(half(o2, 0), cw_r.at[2])
        wait_recv(half(o2, 1), ccw_r.at[2])
        chunk_gemm(o2)

        wait_recv(half(o2p, 0), cw_r.at[3])
        wait_recv(half(o2p, 1), ccw_r.at[3])
        chunk_gemm(o2p)

        for rdma in sends:
            rdma.wait_send()

    return pl.pallas_call(
        body,
        out_shape=jax.ShapeDtypeStruct((N_DEV * m_per, n_per), jnp.float32),
        in_specs=[
            pl.BlockSpec(memory_space=pltpu.VMEM),
            pl.BlockSpec(memory_space=pltpu.VMEM),
        ],
        out_specs=pl.BlockSpec(memory_space=pltpu.VMEM),
        scratch_shapes=[
            pltpu.VMEM((N_DEV * m_per, k), jnp.bfloat16),
            pltpu.VMEM((k, n_per), jnp.bfloat16),
            pltpu.SemaphoreType.DMA((3,)),
            pltpu.SemaphoreType.DMA((3,)),
            pltpu.SemaphoreType.DMA((4,)),
            pltpu.SemaphoreType.DMA((4,)),
            pltpu.SemaphoreType.DMA((4,)),
            pltpu.SemaphoreType.DMA((4,)),
        ],
        compiler_params=pltpu.CompilerParams(collective_id=0),
    )(x, w_mat)
